# Initial kernel scaffold; baseline (speedup 1.0000x reference)
import jax
import jax.numpy as jnp
from jax.experimental import pallas as pl


def kernel(x, edge_index, edge_weight, W1, W2, W3):
    raise NotImplementedError("write your pallas kernel here")


# speedup vs baseline: 3.9499x; 3.9499x over previous
import functools

import jax
import jax.numpy as jnp
from jax import lax
from jax.experimental import pallas as pl
from jax.experimental.pallas import tpu as pltpu
from jax.experimental.pallas import tpu_sc as plsc

N = 10000
E = 320000
D_IN = 128
K = 32
KW = 128
NC = 2
NS = 16
NW = NC * NS
CHUNK = 128
CPW = 80
EPW = CHUNK * CPW
EPAD = EPW * NW
ROWS_A = 624
ROWS_B = N - (NS - 1) * ROWS_A

_sc_mesh = plsc.VectorSubcoreMesh(core_axis_name="c", subcore_axis_name="s")


@functools.partial(
    pl.kernel,
    out_type=jax.ShapeDtypeStruct((NC, N, KW), jnp.float32),
    mesh=_sc_mesh,
    scratch_types=[
        pltpu.VMEM((CPW, CHUNK), jnp.int32),
        pltpu.VMEM((CPW, CHUNK), jnp.int32),
        pltpu.VMEM((CPW, CHUNK), jnp.float32),
        pltpu.VMEM((CHUNK, KW), jnp.float32),
        pltpu.VMEM((16, KW), jnp.float32),
        pltpu.VMEM_SHARED((N, KW), jnp.float32),
        pltpu.SemaphoreType.DMA,
    ],
)
def _sc_spmm(sup_hbm, src_hbm, dst_hbm, w_hbm, out_hbm,
             src_v, dst_v, w_v, rows_v, zblk_v, accum, sem):
    c = lax.axis_index("c")
    s = lax.axis_index("s")
    wid = c * NS + s

    pltpu.sync_copy(src_hbm.at[wid], src_v)
    pltpu.sync_copy(dst_hbm.at[wid], dst_v)
    pltpu.sync_copy(w_hbm.at[wid], w_v)

    zv = jnp.zeros((16,), jnp.float32)
    for r in range(16):
        for q in range(KW // 16):
            zblk_v[r, pl.ds(q * 16, 16)] = zv
    off = pl.multiple_of(s * ROWS_A, 8)

    def zero_body(i, carry):
        pltpu.sync_copy(zblk_v, accum.at[pl.ds(pl.multiple_of(off + i * 16, 8), 16)])
        return carry

    nzero = jnp.where(s < NS - 1, ROWS_A // 16, ROWS_B // 16)
    lax.fori_loop(0, nzero, zero_body, 0)
    plsc.subcore_barrier()

    def chunk_body(j, carry):
        pltpu.async_copy(sup_hbm.at[src_v.at[j]], rows_v, sem).wait()

        def scale_body(g, carry2):
            wv = w_v[j, pl.ds(g * 16, 16)]
            base = g * 16
            for l in range(16):
                w = wv[l]
                rows_v[base + l, pl.ds(0, 16)] = (
                    rows_v[base + l, pl.ds(0, 16)] * w)
                rows_v[base + l, pl.ds(16, 16)] = (
                    rows_v[base + l, pl.ds(16, 16)] * w)
            return carry2

        lax.fori_loop(0, CHUNK // 16, scale_body, 0)

        pltpu.sync_copy(rows_v, accum.at[dst_v.at[j]], add=True)
        return carry

    lax.fori_loop(0, CPW, chunk_body, 0)
    plsc.subcore_barrier()

    @pl.when(s < NS - 1)
    def _():
        pltpu.sync_copy(accum.at[pl.ds(off, ROWS_A)],
                        out_hbm.at[c, pl.ds(off, ROWS_A)])

    @pl.when(s == NS - 1)
    def _():
        pltpu.sync_copy(accum.at[pl.ds(off, ROWS_B)],
                        out_hbm.at[c, pl.ds(off, ROWS_B)])


def _mm1_body(x_ref, w_ref, o_ref):
    h = jnp.dot(x_ref[...], w_ref[...], preferred_element_type=jnp.float32)
    o_ref[...] = jnp.concatenate(
        [h, jnp.zeros((h.shape[0], KW - K), jnp.float32)], axis=1)


def _tc_mm1(x, w1):
    bm = 2000
    return pl.pallas_call(
        _mm1_body,
        grid=(N // bm,),
        in_specs=[pl.BlockSpec((bm, D_IN), lambda i: (i, 0)),
                  pl.BlockSpec((D_IN, K), lambda i: (0, 0))],
        out_specs=pl.BlockSpec((bm, KW), lambda i: (i, 0)),
        out_shape=jax.ShapeDtypeStruct((N, KW), jnp.float32),
    )(x, w1)


def _cmb_body(p0_ref, p1_ref, w_ref, o_ref):
    h = jnp.maximum(p0_ref[:, :K] + p1_ref[:, :K], 0.0)
    h = jnp.dot(h, w_ref[...], preferred_element_type=jnp.float32)
    o_ref[...] = jnp.concatenate(
        [h, jnp.zeros((h.shape[0], KW - K), jnp.float32)], axis=1)


def _tc_relu_mm(p0, p1, w23):
    bm = 2000
    return pl.pallas_call(
        _cmb_body,
        grid=(N // bm,),
        in_specs=[pl.BlockSpec((bm, KW), lambda i: (i, 0)),
                  pl.BlockSpec((bm, KW), lambda i: (i, 0)),
                  pl.BlockSpec((K, K), lambda i: (0, 0))],
        out_specs=pl.BlockSpec((bm, KW), lambda i: (i, 0)),
        out_shape=jax.ShapeDtypeStruct((N, KW), jnp.float32),
    )(p0, p1, w23)


def _add_body(a_ref, b_ref, o_ref):
    o_ref[...] = a_ref[:, :K] + b_ref[:, :K]


def _tc_add(a, b):
    bm = 2000
    return pl.pallas_call(
        _add_body,
        grid=(N // bm,),
        in_specs=[pl.BlockSpec((bm, KW), lambda i: (i, 0)),
                  pl.BlockSpec((bm, KW), lambda i: (i, 0))],
        out_specs=pl.BlockSpec((bm, K), lambda i: (i, 0)),
        out_shape=jax.ShapeDtypeStruct((N, K), jnp.float32),
    )(a, b)


def _dec_body(a_ref, bt_ref, o_ref):
    o_ref[...] = jax.nn.sigmoid(
        jnp.dot(a_ref[...], bt_ref[...], preferred_element_type=jnp.float32))


def _tc_decoder(mu, mu_t):
    bm = 1024
    bn = 1024
    return pl.pallas_call(
        _dec_body,
        grid=(pl.cdiv(N, bm), pl.cdiv(N, bn)),
        in_specs=[pl.BlockSpec((bm, 16), lambda i, j: (i, 0)),
                  pl.BlockSpec((16, bn), lambda i, j: (0, j))],
        out_specs=pl.BlockSpec((bm, bn), lambda i, j: (i, j)),
        out_shape=jax.ShapeDtypeStruct((N, N), jnp.float32),
    )(mu, mu_t)


def kernel(x, edge_index, edge_weight, W1, W2, W3):
    src = edge_index[0]
    dst = edge_index[1]
    pad = EPAD - E
    src3 = jnp.pad(src, (0, pad)).reshape(NW, CPW, CHUNK)
    dst3 = jnp.pad(dst, (0, pad)).reshape(NW, CPW, CHUNK)
    w3 = jnp.pad(edge_weight, (0, pad)).reshape(NW, CPW, CHUNK)

    sup1 = _tc_mm1(x, W1)
    p = _sc_spmm(sup1, src3, dst3, w3)
    w23 = jnp.concatenate([W2, W3], axis=1)
    sup23 = _tc_relu_mm(p[0], p[1], w23)
    q = _sc_spmm(sup23, src3, dst3, w3)
    z = _tc_add(q[0], q[1])
    mu = z[:, :16]
    logvar = z[:, 16:]
    adj = _tc_decoder(mu, mu.T)
    return adj, mu, logvar


# speedup vs baseline: 4.3376x; 1.0981x over previous
import functools

import jax
import jax.numpy as jnp
from jax import lax
from jax.experimental import pallas as pl
from jax.experimental.pallas import tpu as pltpu
from jax.experimental.pallas import tpu_sc as plsc

N = 10000
E = 320000
D_IN = 128
K = 32
NC = 2
NS = 16
NW = NC * NS
CHUNK = 128
CPW = 80
EPW = CHUNK * CPW
EPAD = EPW * NW
ROWS_A = 624
ROWS_B = N - (NS - 1) * ROWS_A
NBUF = 2
HALF = CPW // 2
KW = 128

_sc_mesh = plsc.VectorSubcoreMesh(core_axis_name="c", subcore_axis_name="s")


@functools.partial(
    pl.kernel,
    out_type=jax.ShapeDtypeStruct((NC, N, KW), jnp.float32),
    mesh=_sc_mesh,
    scratch_types=[
        pltpu.VMEM((HALF, CHUNK), jnp.int32),
        pltpu.VMEM((HALF, CHUNK), jnp.int32),
        pltpu.VMEM((HALF, CHUNK), jnp.float32),
        [pltpu.VMEM((CHUNK, KW), jnp.float32) for _ in range(NBUF)],
        pltpu.VMEM((16, KW), jnp.float32),
        pltpu.VMEM_SHARED((N, KW), jnp.float32),
        [pltpu.SemaphoreType.DMA for _ in range(NBUF)],
        [pltpu.SemaphoreType.DMA for _ in range(NBUF)],
    ],
)
def _sc_spmm(sup_hbm, src_hbm, dst_hbm, w_hbm, out_hbm,
             src_v, dst_v, w_v, rows, zblk_v, accum, gsem, ssem):
    c = lax.axis_index("c")
    s = lax.axis_index("s")
    wid = c * NS + s

    off = pl.multiple_of(s * ROWS_A, 8)

    zv = jnp.zeros((16,), jnp.float32)
    for r in range(16):
        for q in range(KW // 16):
            zblk_v[r, pl.ds(q * 16, 16)] = zv

    def zero_body(i, carry):
        pltpu.sync_copy(
            zblk_v, accum.at[pl.ds(pl.multiple_of(off + i * 16, 8), 16)])
        return carry

    nzero = jnp.where(s < NS - 1, ROWS_A // 16, ROWS_B // 16)
    lax.fori_loop(0, nzero, zero_body, 0)
    plsc.subcore_barrier()

    def _scale(j, buf):
        def scale_body(g, carry2):
            wv = w_v[j, pl.ds(g * 16, 16)]
            base = g * 16
            for l in range(16):
                w = wv[l]
                buf[base + l, pl.ds(0, 16)] = buf[base + l, pl.ds(0, 16)] * w
                buf[base + l, pl.ds(16, 16)] = buf[base + l, pl.ds(16, 16)] * w
            return carry2

        lax.fori_loop(0, CHUNK // 16, scale_body, 0)

    for h in range(2):
        pltpu.sync_copy(src_hbm.at[wid, pl.ds(h * HALF, HALF)], src_v)
        pltpu.sync_copy(dst_hbm.at[wid, pl.ds(h * HALF, HALF)], dst_v)
        pltpu.sync_copy(w_hbm.at[wid, pl.ds(h * HALF, HALF)], w_v)

        for b in range(NBUF):
            pltpu.async_copy(sup_hbm.at[src_v.at[b]], rows[b], gsem[b])

        def chunk_body(jj, carry):
            for b in range(NBUF):
                j = jj * NBUF + b
                pltpu.make_async_copy(sup_hbm.at[src_v.at[j]], rows[b],
                                      gsem[b]).wait()
                _scale(j, rows[b])
                pltpu.async_copy(rows[b], accum.at[dst_v.at[j]], ssem[b],
                                 add=True)

            @pl.when(jj < HALF // NBUF - 1)
            def _():
                for b in range(NBUF):
                    j = jj * NBUF + b
                    pltpu.make_async_copy(rows[b], accum.at[dst_v.at[j]],
                                          ssem[b]).wait()
                    pltpu.async_copy(sup_hbm.at[src_v.at[j + NBUF]], rows[b],
                                     gsem[b])

            return carry

        lax.fori_loop(0, HALF // NBUF, chunk_body, 0)
        for b in range(NBUF):
            j = HALF - NBUF + b
            pltpu.make_async_copy(rows[b], accum.at[dst_v.at[j]],
                                  ssem[b]).wait()
    plsc.subcore_barrier()

    @pl.when(s < NS - 1)
    def _():
        pltpu.sync_copy(accum.at[pl.ds(off, ROWS_A)],
                        out_hbm.at[c, pl.ds(off, ROWS_A)])

    @pl.when(s == NS - 1)
    def _():
        pltpu.sync_copy(accum.at[pl.ds(off, ROWS_B)],
                        out_hbm.at[c, pl.ds(off, ROWS_B)])


def _mm1_body(x_ref, w_ref, o_ref):
    h = jnp.dot(x_ref[...], w_ref[...], preferred_element_type=jnp.float32)
    o_ref[...] = jnp.concatenate(
        [h, jnp.zeros((h.shape[0], KW - K), jnp.float32)], axis=1)


def _tc_mm1(x, w1):
    bm = 2000
    return pl.pallas_call(
        _mm1_body,
        grid=(N // bm,),
        in_specs=[pl.BlockSpec((bm, D_IN), lambda i: (i, 0)),
                  pl.BlockSpec((D_IN, K), lambda i: (0, 0))],
        out_specs=pl.BlockSpec((bm, KW), lambda i: (i, 0)),
        out_shape=jax.ShapeDtypeStruct((N, KW), jnp.float32),
    )(x, w1)


def _cmb_body(p0_ref, p1_ref, w_ref, o_ref):
    h = jnp.maximum(p0_ref[:, :K] + p1_ref[:, :K], 0.0)
    h = jnp.dot(h, w_ref[...], preferred_element_type=jnp.float32)
    o_ref[...] = jnp.concatenate(
        [h, jnp.zeros((h.shape[0], KW - K), jnp.float32)], axis=1)


def _tc_relu_mm(p0, p1, w23):
    bm = 2000
    return pl.pallas_call(
        _cmb_body,
        grid=(N // bm,),
        in_specs=[pl.BlockSpec((bm, KW), lambda i: (i, 0)),
                  pl.BlockSpec((bm, KW), lambda i: (i, 0)),
                  pl.BlockSpec((K, K), lambda i: (0, 0))],
        out_specs=pl.BlockSpec((bm, KW), lambda i: (i, 0)),
        out_shape=jax.ShapeDtypeStruct((N, KW), jnp.float32),
    )(p0, p1, w23)


def _add_body(a_ref, b_ref, o_ref):
    o_ref[...] = a_ref[:, :K] + b_ref[:, :K]


def _tc_add(a, b):
    bm = 2000
    return pl.pallas_call(
        _add_body,
        grid=(N // bm,),
        in_specs=[pl.BlockSpec((bm, KW), lambda i: (i, 0)),
                  pl.BlockSpec((bm, KW), lambda i: (i, 0))],
        out_specs=pl.BlockSpec((bm, K), lambda i: (i, 0)),
        out_shape=jax.ShapeDtypeStruct((N, K), jnp.float32),
    )(a, b)


def _dec_body(a_ref, bt_ref, o_ref):
    o_ref[...] = jax.nn.sigmoid(
        jnp.dot(a_ref[...], bt_ref[...], preferred_element_type=jnp.float32))


def _tc_decoder(mu, mu_t):
    bm = 1024
    bn = 1024
    return pl.pallas_call(
        _dec_body,
        grid=(pl.cdiv(N, bm), pl.cdiv(N, bn)),
        in_specs=[pl.BlockSpec((bm, 16), lambda i, j: (i, 0)),
                  pl.BlockSpec((16, bn), lambda i, j: (0, j))],
        out_specs=pl.BlockSpec((bm, bn), lambda i, j: (i, j)),
        out_shape=jax.ShapeDtypeStruct((N, N), jnp.float32),
    )(mu, mu_t)


def kernel(x, edge_index, edge_weight, W1, W2, W3):
    src = edge_index[0]
    dst = edge_index[1]
    pad = EPAD - E
    src3 = jnp.pad(src, (0, pad)).reshape(NW, CPW, CHUNK)
    dst3 = jnp.pad(dst, (0, pad)).reshape(NW, CPW, CHUNK)
    w3 = jnp.pad(edge_weight, (0, pad)).reshape(NW, CPW, CHUNK)

    sup1 = _tc_mm1(x, W1)
    p = _sc_spmm(sup1, src3, dst3, w3)
    w23 = jnp.concatenate([W2, W3], axis=1)
    sup23 = _tc_relu_mm(p[0], p[1], w23)
    q = _sc_spmm(sup23, src3, dst3, w3)
    z = _tc_add(q[0], q[1])
    mu = z[:, :16]
    logvar = z[:, 16:]
    adj = _tc_decoder(mu, mu.T)
    return adj, mu, logvar
